# baseline (speedup 1.0000x reference)
import functools

import jax
import jax.numpy as jnp
from jax import lax
from jax.experimental import pallas as pl
from jax.experimental.pallas import tpu as pltpu
from jax.experimental.pallas import tpu_sc as plsc

_B = 16384
_D = 64
_LANES = 16

_INFO = plsc.get_sparse_core_info()
_NC = _INFO.num_cores
_NS = _INFO.num_subcores
_NW = _NC * _NS
_BPW = _B // _NW
_CHUNK = 512
_NCHUNK = _BPW // _CHUNK


def _body(idx_hbm, single_hbm, cat_hbm, out_hbm,
          idxp_v, idxc_v, rows_a, rows_b,
          sem_idx, sem_w, *sems_ab):
    sems_a = sems_ab[:_NCHUNK]
    sems_b = sems_ab[_NCHUNK:]
    wid = lax.axis_index("s") * _NC + lax.axis_index("c")
    base = wid * _BPW

    cp0 = pltpu.async_copy(idx_hbm.at[pl.ds(base, _BPW)], idxp_v, sem_idx)
    cp1 = pltpu.async_copy(idx_hbm.at[pl.ds(_B + base, _BPW)], idxc_v, sem_idx)
    cp0.wait()
    cp1.wait()

    gathers = []
    for j in range(_NCHUNK):
        sl = pl.ds(j * _CHUNK, _CHUNK)
        gathers.append((
            pltpu.async_copy(single_hbm.at[idxp_v.at[sl]], rows_a.at[sl],
                             sems_a[j]),
            pltpu.async_copy(cat_hbm.at[idxc_v.at[sl]], rows_b.at[sl],
                             sems_b[j]),
        ))

    writes = []
    for j in range(_NCHUNK):
        ga, gb = gathers[j]
        ga.wait()
        gb.wait()

        @plsc.parallel_loop(j * _CHUNK, (j + 1) * _CHUNK, unroll=4)
        def add_row(i):
            for t in range(_D // _LANES):
                sl = pl.ds(t * _LANES, _LANES)
                rows_a[i, sl] = rows_a[i, sl] + rows_b[i, sl]

        writes.append(pltpu.async_copy(
            rows_a.at[pl.ds(j * _CHUNK, _CHUNK)],
            out_hbm.at[pl.ds(base + j * _CHUNK, _CHUNK), pl.ds(0, _D)],
            sem_w))
    for cp in writes:
        cp.wait()


@jax.jit
def _poi_embedding(idx_flat, single_embeddings, category_embeddings):
    mesh = plsc.VectorSubcoreMesh(core_axis_name="c", subcore_axis_name="s")
    kfn = pl.kernel(
        _body,
        out_type=jax.ShapeDtypeStruct((_B, 128), jnp.float32),
        mesh=mesh,
        scratch_types=[
            pltpu.VMEM((_BPW,), jnp.int32),
            pltpu.VMEM((_BPW,), jnp.int32),
            pltpu.VMEM((_BPW, _D), jnp.float32),
            pltpu.VMEM((_BPW, _D), jnp.float32),
            pltpu.SemaphoreType.DMA,
            pltpu.SemaphoreType.DMA,
        ] + [pltpu.SemaphoreType.DMA] * (2 * _NCHUNK),
        compiler_params=pltpu.CompilerParams(use_tc_tiling_on_sc=False),
    )
    return kfn(idx_flat, single_embeddings, category_embeddings)


def kernel(poi_vec, category_embeddings, single_embeddings):
    idx_flat = poi_vec.T.reshape(2 * _B)
    out_wide = _poi_embedding(idx_flat, single_embeddings, category_embeddings)
    return out_wide[:, :_D]


# speedup vs baseline: 1.0027x; 1.0027x over previous
import jax
import jax.numpy as jnp
from jax import lax
from jax.experimental import pallas as pl
from jax.experimental.pallas import tpu as pltpu
from jax.experimental.pallas import tpu_sc as plsc

_B = 16384
_D = 64
_LANES = 16

_INFO = plsc.get_sparse_core_info()
_NC = _INFO.num_cores
_NS = _INFO.num_subcores
_NW = _NC * _NS
_BPW = _B // _NW
_CHUNK = 256
_NCHUNK = _BPW // _CHUNK


def _body(idx_hbm, single_hbm, cat_hbm, out_hbm,
          idxp_v, idxc_v, rows_a, rows_b,
          sem_idx, sem_w, *sems_ab):
    sems_a = sems_ab[:_NCHUNK]
    sems_b = sems_ab[_NCHUNK:]
    wid = lax.axis_index("s") * _NC + lax.axis_index("c")
    base = wid * _BPW

    cp0 = pltpu.async_copy(idx_hbm.at[pl.ds(base, _BPW)], idxp_v, sem_idx)
    cp1 = pltpu.async_copy(idx_hbm.at[pl.ds(_B + base, _BPW)], idxc_v, sem_idx)
    cp0.wait()
    cp1.wait()

    gathers = []
    for j in range(_NCHUNK):
        sl = pl.ds(j * _CHUNK, _CHUNK)
        gathers.append((
            pltpu.async_copy(single_hbm.at[idxp_v.at[sl]], rows_a.at[sl],
                             sems_a[j]),
            pltpu.async_copy(cat_hbm.at[idxc_v.at[sl]], rows_b.at[sl],
                             sems_b[j]),
        ))

    writes = []
    for j in range(_NCHUNK):
        ga, gb = gathers[j]
        ga.wait()
        gb.wait()

        @plsc.parallel_loop(j * _CHUNK, (j + 1) * _CHUNK, unroll=4)
        def add_row(i):
            for t in range(_D // _LANES):
                sl = pl.ds(t * _LANES, _LANES)
                rows_a[i, sl] = rows_a[i, sl] + rows_b[i, sl]

        writes.append(pltpu.async_copy(
            rows_a.at[pl.ds(j * _CHUNK, _CHUNK)],
            out_hbm.at[pl.ds(base + j * _CHUNK, _CHUNK), pl.ds(0, _D)],
            sem_w))
    for cp in writes:
        cp.wait()


@jax.jit
def _poi_embedding(idx_flat, single_embeddings, category_embeddings):
    mesh = plsc.VectorSubcoreMesh(core_axis_name="c", subcore_axis_name="s")
    kfn = pl.kernel(
        _body,
        out_type=jax.ShapeDtypeStruct((_B, 128), jnp.float32),
        mesh=mesh,
        scratch_types=[
            pltpu.VMEM((_BPW,), jnp.int32),
            pltpu.VMEM((_BPW,), jnp.int32),
            pltpu.VMEM((_BPW, _D), jnp.float32),
            pltpu.VMEM((_BPW, _D), jnp.float32),
            pltpu.SemaphoreType.DMA,
            pltpu.SemaphoreType.DMA,
        ] + [pltpu.SemaphoreType.DMA] * (2 * _NCHUNK),
        compiler_params=pltpu.CompilerParams(use_tc_tiling_on_sc=False),
    )
    return kfn(idx_flat, single_embeddings, category_embeddings)


def kernel(poi_vec, category_embeddings, single_embeddings):
    idx_flat = poi_vec.T.reshape(2 * _B)
    out_wide = _poi_embedding(idx_flat, single_embeddings, category_embeddings)
    return out_wide[:, :_D]
